# Initial kernel scaffold; baseline (speedup 1.0000x reference)
import jax
import jax.numpy as jnp
from jax.experimental import pallas as pl


def kernel(x_user, x_item, edge_index_ui, edge_index_iu, proj_user_W, proj_user_b, proj_item_W, proj_item_b, l0_ui_Wl, l0_ui_bl, l0_ui_Wr, l0_iu_Wl, l0_iu_bl, l0_iu_Wr, ln0_user_g, ln0_user_b, ln0_item_g, ln0_item_b, l1_ui_Wl, l1_ui_bl, l1_ui_Wr, l1_iu_Wl, l1_iu_bl, l1_iu_Wr, ln1_user_g, ln1_user_b, ln1_item_g, ln1_item_b):
    raise NotImplementedError("write your pallas kernel here")


# speedup vs baseline: 5.8220x; 5.8220x over previous
import functools

import jax
import jax.numpy as jnp
from jax import lax
from jax.experimental import pallas as pl
from jax.experimental.pallas import tpu as pltpu
from jax.experimental.pallas import tpu_sc as plsc

N = 50000
E = 800000
D_IN = 128
H = 64
OUT = 32
EPS = 1e-5

W = 128
NQP = 6252
EP = NQP * W
PAD = EP - E
NS = 16
NC = 2
NA = N + 8
STR = 3128

_f32 = jnp.float32


_SC_PARAMS = pltpu.CompilerParams(use_tc_tiling_on_sc=False)
_MESH = plsc.VectorSubcoreMesh(core_axis_name="c", subcore_axis_name="s")


def _zero_acc(zrows_ref, acc, s):
  @pl.when(s < NS - 1)
  def _():
    pltpu.sync_copy(zrows_ref, acc.at[pl.ds(s * STR, STR)])

  @pl.when(s == NS - 1)
  def _():
    last = NA - (NS - 1) * STR
    pltpu.sync_copy(zrows_ref.at[pl.ds(0, last)],
                    acc.at[pl.ds((NS - 1) * STR, last)])


def _dump_acc(acc, out2d, s):
  @pl.when(s < NS - 1)
  def _():
    pltpu.sync_copy(acc.at[pl.ds(s * STR, STR)], out2d.at[pl.ds(s * STR, STR)])

  @pl.when(s == NS - 1)
  def _():
    last = N - (NS - 1) * STR
    pltpu.sync_copy(acc.at[pl.ds((NS - 1) * STR, last)],
                    out2d.at[pl.ds((NS - 1) * STR, last)])


def _make_segsum(d2, k):
  nsup = NQP // k
  out_type = [jax.ShapeDtypeStruct((NC, N, d2), _f32)]
  scratch = [
      pltpu.VMEM((k, W), jnp.int32),
      pltpu.VMEM((k, W), jnp.int32),
      pltpu.VMEM((k * W, d2), _f32),
      pltpu.VMEM_SHARED((NA, d2), _f32),
      pltpu.SemaphoreType.DMA,
      pltpu.SemaphoreType.DMA,
  ]

  def body(table_ref, src2_ref, dst_ref, zrows_ref, out_ref,
           srcv, dstv, rows, acc, semg, sems):
    c = lax.axis_index("c")
    s = lax.axis_index("s")
    _zero_acc(zrows_ref, acc, s)
    plsc.subcore_barrier()

    nt = (nsup - s + NS - 1) // NS

    def step(t, carry):
      q = s + t * NS
      pltpu.sync_copy(src2_ref.at[c].at[pl.ds(q * k, k)], srcv)
      pltpu.sync_copy(dst_ref.at[pl.ds(q * k, k)], dstv)
      gs = [
          pltpu.make_async_copy(
              table_ref.at[srcv.at[j]], rows.at[pl.ds(j * W, W)], semg)
          for j in range(k)
      ]
      for g in gs:
        g.start()
      sc = [
          pltpu.make_async_copy(
              rows.at[pl.ds(j * W, W)], acc.at[dstv.at[j]], sems)
          for j in range(k)
      ]
      for g, x in zip(gs, sc):
        g.wait()
        x.start(add=True)
      for x in sc:
        x.wait()
      return carry

    lax.fori_loop(0, nt, step, 0)
    plsc.subcore_barrier()
    _dump_acc(acc, out_ref.at[c], s)

  return pl.kernel(
      body, out_type=out_type, mesh=_MESH, scratch_types=scratch,
      compiler_params=_SC_PARAMS)


_segsum_h = _make_segsum(H // 2, 4)
_segsum_o = _make_segsum(OUT // 2, 6)

_KC = 12
_CW = 16


def _cnt_body(dst2_ref, zcnt_ref, ones_ref, out_ref, dstv, onesv, cntacc,
              semc):
  c = lax.axis_index("c")
  s = lax.axis_index("s")

  @pl.when(s < NS - 1)
  def _():
    pltpu.sync_copy(zcnt_ref, cntacc.at[pl.ds(s * STR, STR)])

  @pl.when(s == NS - 1)
  def _():
    last = NA - (NS - 1) * STR
    pltpu.sync_copy(zcnt_ref.at[pl.ds(0, last)],
                    cntacc.at[pl.ds((NS - 1) * STR, last)])

  pltpu.sync_copy(ones_ref, onesv)
  plsc.subcore_barrier()

  nsup = NQP // _KC

  nt = (nsup - s + NS - 1) // NS

  def step(t, carry):
    q = s + t * NS
    pltpu.sync_copy(dst2_ref.at[c].at[pl.ds(q * _KC, _KC)], dstv)
    cs = [
        pltpu.make_async_copy(onesv, cntacc.at[dstv.at[j]], semc)
        for j in range(_KC)
    ]
    for x in cs:
      x.start(add=True)
    for x in cs:
      x.wait()
    return carry

  lax.fori_loop(0, nt, step, 0)
  plsc.subcore_barrier()
  _dump_acc(cntacc, out_ref.at[c], s)


_cnt_kernel = pl.kernel(
    _cnt_body,
    out_type=[jax.ShapeDtypeStruct((NC, N, _CW), _f32)],
    mesh=_MESH,
    scratch_types=[
        pltpu.VMEM((_KC, W), jnp.int32),
        pltpu.VMEM((W, _CW), _f32),
        pltpu.VMEM_SHARED((NA, _CW), _f32),
        pltpu.SemaphoreType.DMA,
    ],
    compiler_params=_SC_PARAMS)


R = 1000
NB = N // R

_PREC = lax.Precision.HIGHEST


def _dot(a, b):
  return jnp.dot(a, b, precision=_PREC, preferred_element_type=_f32)


def _ln(h, g, b):
  m = jnp.mean(h, axis=-1, keepdims=True)
  v = jnp.mean((h - m) * (h - m), axis=-1, keepdims=True)
  return (h - m) * lax.rsqrt(v + EPS) * g + b


def _row_spec(d):
  return pl.BlockSpec((R, d), lambda i: (i, 0))


def _split_spec(d):
  return pl.BlockSpec((2, R, d), lambda i: (0, i, 0))


def _w_spec(a, b):
  return pl.BlockSpec((a, b), lambda i: (0, 0))


def _proj_body(xu_ref, xi_ref, puW_ref, pub_ref, piW_ref, pib_ref,
               hu_ref, hi_ref):
  hu_ref[...] = jnp.maximum(_dot(xu_ref[...], puW_ref[...]) + pub_ref[...],
                            0.0)
  hi_ref[...] = jnp.maximum(_dot(xi_ref[...], piW_ref[...]) + pib_ref[...],
                            0.0)


_proj = pl.pallas_call(
    _proj_body,
    grid=(NB,),
    in_specs=[
        _row_spec(D_IN), _row_spec(D_IN),
        _w_spec(D_IN, H), _w_spec(1, H),
        _w_spec(D_IN, H), _w_spec(1, H),
    ],
    out_specs=[_row_spec(H), _row_spec(H)],
    out_shape=[
        jax.ShapeDtypeStruct((N, H), _f32),
        jax.ShapeDtypeStruct((N, H), _f32),
    ],
)


def _l0_half(s2_ref, cnt_ref, h_ref, Wl_ref, bl_ref, Wr_ref, g_ref, b_ref,
             Wnext_ref, h1_ref, m_ref):
  inv = 1.0 / jnp.maximum(cnt_ref[...], 1.0)
  agg = jnp.concatenate([s2_ref[0], s2_ref[1]], axis=-1) * inv
  o = _dot(agg, Wl_ref[...]) + bl_ref[...] + _dot(h_ref[...], Wr_ref[...])
  h1 = jnp.maximum(_ln(o, g_ref[...], b_ref[...]), 0.0)
  h1_ref[...] = h1
  m_ref[...] = _dot(h1, Wnext_ref[...])


def _l0_body(su2_ref, cntu_ref, hu_ref, iuWl_ref, iubl_ref, iuWr_ref,
             n0ug_ref, n0ub_ref, uiWl1_ref,
             si2_ref, cnti_ref, hi_ref, uiWl_ref, uibl_ref, uiWr_ref,
             n0ig_ref, n0ib_ref, iuWl1_ref,
             h1u_ref, mu_ref, h1i_ref, mi_ref):
  _l0_half(su2_ref, cntu_ref, hu_ref, iuWl_ref, iubl_ref, iuWr_ref,
           n0ug_ref, n0ub_ref, uiWl1_ref, h1u_ref, mu_ref)
  _l0_half(si2_ref, cnti_ref, hi_ref, uiWl_ref, uibl_ref, uiWr_ref,
           n0ig_ref, n0ib_ref, iuWl1_ref, h1i_ref, mi_ref)


_l0_post = pl.pallas_call(
    _l0_body,
    grid=(NB,),
    in_specs=[
        _split_spec(H // 2), _row_spec(1), _row_spec(H),
        _w_spec(H, H), _w_spec(1, H), _w_spec(H, H),
        _w_spec(1, H), _w_spec(1, H), _w_spec(H, OUT),
        _split_spec(H // 2), _row_spec(1), _row_spec(H),
        _w_spec(H, H), _w_spec(1, H), _w_spec(H, H),
        _w_spec(1, H), _w_spec(1, H), _w_spec(H, OUT),
    ],
    out_specs=[
        _row_spec(H), _row_spec(OUT),
        _row_spec(H), _row_spec(OUT),
    ],
    out_shape=[
        jax.ShapeDtypeStruct((N, H), _f32),
        jax.ShapeDtypeStruct((N, OUT), _f32),
        jax.ShapeDtypeStruct((N, H), _f32),
        jax.ShapeDtypeStruct((N, OUT), _f32),
    ],
)


def _l1_half(s2_ref, cnt_ref, h1_ref, Wr_ref, bl_ref, g_ref, b_ref, out_ref):
  inv = 1.0 / jnp.maximum(cnt_ref[...], 1.0)
  agg = jnp.concatenate([s2_ref[0], s2_ref[1]], axis=-1) * inv
  o = agg + bl_ref[...] + _dot(h1_ref[...], Wr_ref[...])
  out_ref[...] = _ln(o, g_ref[...], b_ref[...])


def _l1_body(su2_ref, cntu_ref, h1u_ref, iuWr_ref, iubl_ref, n1ug_ref,
             n1ub_ref,
             si2_ref, cnti_ref, h1i_ref, uiWr_ref, uibl_ref, n1ig_ref,
             n1ib_ref,
             outu_ref, outi_ref):
  _l1_half(su2_ref, cntu_ref, h1u_ref, iuWr_ref, iubl_ref, n1ug_ref,
           n1ub_ref, outu_ref)
  _l1_half(si2_ref, cnti_ref, h1i_ref, uiWr_ref, uibl_ref, n1ig_ref,
           n1ib_ref, outi_ref)


_l1_post = pl.pallas_call(
    _l1_body,
    grid=(NB,),
    in_specs=[
        _split_spec(OUT // 2), _row_spec(1), _row_spec(H),
        _w_spec(H, OUT), _w_spec(1, OUT), _w_spec(1, OUT), _w_spec(1, OUT),
        _split_spec(OUT // 2), _row_spec(1), _row_spec(H),
        _w_spec(H, OUT), _w_spec(1, OUT), _w_spec(1, OUT), _w_spec(1, OUT),
    ],
    out_specs=[_row_spec(OUT), _row_spec(OUT)],
    out_shape=[
        jax.ShapeDtypeStruct((N, OUT), _f32),
        jax.ShapeDtypeStruct((N, OUT), _f32),
    ],
)


@jax.jit
def kernel(x_user, x_item, edge_index_ui, edge_index_iu, proj_user_W,
           proj_user_b, proj_item_W, proj_item_b, l0_ui_Wl, l0_ui_bl,
           l0_ui_Wr, l0_iu_Wl, l0_iu_bl, l0_iu_Wr, ln0_user_g, ln0_user_b,
           ln0_item_g, ln0_item_b, l1_ui_Wl, l1_ui_bl, l1_ui_Wr, l1_iu_Wl,
           l1_iu_bl, l1_iu_Wr, ln1_user_g, ln1_user_b, ln1_item_g,
           ln1_item_b):
  pad = jnp.arange(PAD, dtype=jnp.int32)
  pad_src = pad % W
  pad_dst = N + (pad % 8)

  def _prep(ei):
    src = jnp.concatenate([ei[0], pad_src])
    dst = jnp.concatenate([ei[1], pad_dst]).reshape(NQP, W)
    src2 = jnp.stack([2 * src, 2 * src + 1]).reshape(NC, NQP, W)
    return src2, dst

  src2_ui, dst_ui = _prep(edge_index_ui)
  src2_iu, dst_iu = _prep(edge_index_iu)
  dst2 = jnp.stack([dst_ui, dst_iu])

  zrows_h = jnp.zeros((STR, H // 2), _f32)
  zrows_o = jnp.zeros((STR, OUT // 2), _f32)
  zcnt = jnp.zeros((STR, _CW), _f32)
  ones = jnp.ones((W, _CW), _f32)

  (cnt2,) = _cnt_kernel(dst2, zcnt, ones)
  cnt_i = cnt2[0, :, 0:1]
  cnt_u = cnt2[1, :, 0:1]

  hu, hi = _proj(x_user, x_item, proj_user_W, proj_user_b[None, :],
                 proj_item_W, proj_item_b[None, :])

  (sums_i2,) = _segsum_h(hu.reshape(NC * N, H // 2), src2_ui, dst_ui,
                         zrows_h)
  (sums_u2,) = _segsum_h(hi.reshape(NC * N, H // 2), src2_iu, dst_iu,
                         zrows_h)

  h1u, mu, h1i, mi = _l0_post(
      sums_u2, cnt_u, hu, l0_iu_Wl, l0_iu_bl[None, :], l0_iu_Wr,
      ln0_user_g[None, :], ln0_user_b[None, :], l1_ui_Wl,
      sums_i2, cnt_i, hi, l0_ui_Wl, l0_ui_bl[None, :], l0_ui_Wr,
      ln0_item_g[None, :], ln0_item_b[None, :], l1_iu_Wl)

  (s32_i2,) = _segsum_o(mu.reshape(NC * N, OUT // 2), src2_ui, dst_ui,
                        zrows_o)
  (s32_u2,) = _segsum_o(mi.reshape(NC * N, OUT // 2), src2_iu, dst_iu,
                        zrows_o)

  out_u, out_i = _l1_post(
      s32_u2, cnt_u, h1u, l1_iu_Wr, l1_iu_bl[None, :],
      ln1_user_g[None, :], ln1_user_b[None, :],
      s32_i2, cnt_i, h1i, l1_ui_Wr, l1_ui_bl[None, :],
      ln1_item_g[None, :], ln1_item_b[None, :])
  return out_u, out_i


# speedup vs baseline: 7.0403x; 1.2093x over previous
import functools

import jax
import jax.numpy as jnp
from jax import lax
from jax.experimental import pallas as pl
from jax.experimental.pallas import tpu as pltpu
from jax.experimental.pallas import tpu_sc as plsc

N = 50000
E = 800000
D_IN = 128
H = 64
OUT = 32
EPS = 1e-5

W = 128
NQP = 6252
EP = NQP * W
PAD = EP - E
NS = 16
NC = 2
NA = N + 8
STR = 3128

_f32 = jnp.float32


_SC_PARAMS = pltpu.CompilerParams(use_tc_tiling_on_sc=False)
_MESH = plsc.VectorSubcoreMesh(core_axis_name="c", subcore_axis_name="s")


def _zero_acc(zrows_ref, acc, s):
  @pl.when(s < NS - 1)
  def _():
    pltpu.sync_copy(zrows_ref, acc.at[pl.ds(s * STR, STR)])

  @pl.when(s == NS - 1)
  def _():
    last = NA - (NS - 1) * STR
    pltpu.sync_copy(zrows_ref.at[pl.ds(0, last)],
                    acc.at[pl.ds((NS - 1) * STR, last)])


def _dump_acc(acc, out2d, s):
  @pl.when(s < NS - 1)
  def _():
    pltpu.sync_copy(acc.at[pl.ds(s * STR, STR)], out2d.at[pl.ds(s * STR, STR)])

  @pl.when(s == NS - 1)
  def _():
    last = N - (NS - 1) * STR
    pltpu.sync_copy(acc.at[pl.ds((NS - 1) * STR, last)],
                    out2d.at[pl.ds((NS - 1) * STR, last)])


def _make_segsum(d2, k):
  nsup = NQP // k
  out_type = [jax.ShapeDtypeStruct((NC, N, d2), _f32)]
  scratch = [
      pltpu.VMEM((2, k, W), jnp.int32),
      pltpu.VMEM((2, k, W), jnp.int32),
      pltpu.VMEM((2, k * W, d2), _f32),
      pltpu.VMEM_SHARED((NA, d2), _f32),
      pltpu.SemaphoreType.DMA,
      pltpu.SemaphoreType.DMA,
      pltpu.SemaphoreType.DMA,
  ]

  def body(table_ref, src2_ref, dst_ref, zrows_ref, out_ref,
           srcv, dstv, rows, acc, semi, semg, sems):
    c = lax.axis_index("c")
    s = lax.axis_index("s")
    _zero_acc(zrows_ref, acc, s)
    plsc.subcore_barrier()

    nt = (nsup - s + NS - 1) // NS

    def idx_copies(p, q):
      return [
          pltpu.make_async_copy(src2_ref.at[c].at[pl.ds(q * k, k)],
                                srcv.at[p], semi),
          pltpu.make_async_copy(dst_ref.at[pl.ds(q * k, k)],
                                dstv.at[p], semi),
      ]

    def gather_copies(p):
      return [
          pltpu.make_async_copy(table_ref.at[srcv.at[p].at[j]],
                                rows.at[p].at[pl.ds(j * W, W)], semg)
          for j in range(k)
      ]

    def scatter_copies(p):
      return [
          pltpu.make_async_copy(rows.at[p].at[pl.ds(j * W, W)],
                                acc.at[dstv.at[p].at[j]], sems)
          for j in range(k)
      ]

    @pl.when(nt > 0)
    def _():
      for x in idx_copies(0, s):
        x.start()

    def trip(t, u):
      @pl.when(t < nt)
      def _():
        q = s + t * NS
        for x in idx_copies(u, q):
          x.wait()

        @pl.when(t >= 2)
        def _():
          for x in scatter_copies(u):
            x.wait()

        gs = gather_copies(u)
        for x in gs:
          x.start()

        @pl.when(t + 1 < nt)
        def _():
          for x in idx_copies(1 - u, q + NS):
            x.start()

        sc = scatter_copies(u)
        for g, x in zip(gs, sc):
          g.wait()
          x.start(add=True)

    def pair(r, carry):
      trip(2 * r, 0)
      trip(2 * r + 1, 1)
      return carry

    lax.fori_loop(0, (nt + 1) // 2, pair, 0)

    @pl.when(nt >= 2)
    def _():
      for x in scatter_copies(0) + scatter_copies(1):
        x.wait()

    @pl.when(nt == 1)
    def _():
      for x in scatter_copies(0):
        x.wait()

    plsc.subcore_barrier()
    _dump_acc(acc, out_ref.at[c], s)

  return pl.kernel(
      body, out_type=out_type, mesh=_MESH, scratch_types=scratch,
      compiler_params=_SC_PARAMS)


_segsum_h = _make_segsum(H // 2, 3)
_segsum_o = _make_segsum(OUT // 2, 6)

_KC = 12
_CW = 16


def _cnt_body(dst2_ref, zcnt_ref, ones_ref, out_ref, dstv, onesv, cntacc,
              semc):
  c = lax.axis_index("c")
  s = lax.axis_index("s")

  @pl.when(s < NS - 1)
  def _():
    pltpu.sync_copy(zcnt_ref, cntacc.at[pl.ds(s * STR, STR)])

  @pl.when(s == NS - 1)
  def _():
    last = NA - (NS - 1) * STR
    pltpu.sync_copy(zcnt_ref.at[pl.ds(0, last)],
                    cntacc.at[pl.ds((NS - 1) * STR, last)])

  pltpu.sync_copy(ones_ref, onesv)
  plsc.subcore_barrier()

  nsup = NQP // _KC

  nt = (nsup - s + NS - 1) // NS

  def step(t, carry):
    q = s + t * NS
    pltpu.sync_copy(dst2_ref.at[c].at[pl.ds(q * _KC, _KC)], dstv)
    cs = [
        pltpu.make_async_copy(onesv, cntacc.at[dstv.at[j]], semc)
        for j in range(_KC)
    ]
    for x in cs:
      x.start(add=True)
    for x in cs:
      x.wait()
    return carry

  lax.fori_loop(0, nt, step, 0)
  plsc.subcore_barrier()
  _dump_acc(cntacc, out_ref.at[c], s)


_cnt_kernel = pl.kernel(
    _cnt_body,
    out_type=[jax.ShapeDtypeStruct((NC, N, _CW), _f32)],
    mesh=_MESH,
    scratch_types=[
        pltpu.VMEM((_KC, W), jnp.int32),
        pltpu.VMEM((W, _CW), _f32),
        pltpu.VMEM_SHARED((NA, _CW), _f32),
        pltpu.SemaphoreType.DMA,
    ],
    compiler_params=_SC_PARAMS)


R = 1000
NB = N // R

_PREC = lax.Precision.HIGHEST


def _dot(a, b):
  return jnp.dot(a, b, precision=_PREC, preferred_element_type=_f32)


def _ln(h, g, b):
  m = jnp.mean(h, axis=-1, keepdims=True)
  v = jnp.mean((h - m) * (h - m), axis=-1, keepdims=True)
  return (h - m) * lax.rsqrt(v + EPS) * g + b


def _row_spec(d):
  return pl.BlockSpec((R, d), lambda i: (i, 0))


def _split_spec(d):
  return pl.BlockSpec((2, R, d), lambda i: (0, i, 0))


def _w_spec(a, b):
  return pl.BlockSpec((a, b), lambda i: (0, 0))


def _proj_body(xu_ref, xi_ref, puW_ref, pub_ref, piW_ref, pib_ref,
               hu_ref, hi_ref):
  hu_ref[...] = jnp.maximum(_dot(xu_ref[...], puW_ref[...]) + pub_ref[...],
                            0.0)
  hi_ref[...] = jnp.maximum(_dot(xi_ref[...], piW_ref[...]) + pib_ref[...],
                            0.0)


_proj = pl.pallas_call(
    _proj_body,
    grid=(NB,),
    in_specs=[
        _row_spec(D_IN), _row_spec(D_IN),
        _w_spec(D_IN, H), _w_spec(1, H),
        _w_spec(D_IN, H), _w_spec(1, H),
    ],
    out_specs=[_row_spec(H), _row_spec(H)],
    out_shape=[
        jax.ShapeDtypeStruct((N, H), _f32),
        jax.ShapeDtypeStruct((N, H), _f32),
    ],
)


def _l0_half(s2_ref, cnt_ref, h_ref, Wl_ref, bl_ref, Wr_ref, g_ref, b_ref,
             Wnext_ref, h1_ref, m_ref):
  inv = 1.0 / jnp.maximum(cnt_ref[...], 1.0)
  agg = jnp.concatenate([s2_ref[0], s2_ref[1]], axis=-1) * inv
  o = _dot(agg, Wl_ref[...]) + bl_ref[...] + _dot(h_ref[...], Wr_ref[...])
  h1 = jnp.maximum(_ln(o, g_ref[...], b_ref[...]), 0.0)
  h1_ref[...] = h1
  m_ref[...] = _dot(h1, Wnext_ref[...])


def _l0_body(su2_ref, cntu_ref, hu_ref, iuWl_ref, iubl_ref, iuWr_ref,
             n0ug_ref, n0ub_ref, uiWl1_ref,
             si2_ref, cnti_ref, hi_ref, uiWl_ref, uibl_ref, uiWr_ref,
             n0ig_ref, n0ib_ref, iuWl1_ref,
             h1u_ref, mu_ref, h1i_ref, mi_ref):
  _l0_half(su2_ref, cntu_ref, hu_ref, iuWl_ref, iubl_ref, iuWr_ref,
           n0ug_ref, n0ub_ref, uiWl1_ref, h1u_ref, mu_ref)
  _l0_half(si2_ref, cnti_ref, hi_ref, uiWl_ref, uibl_ref, uiWr_ref,
           n0ig_ref, n0ib_ref, iuWl1_ref, h1i_ref, mi_ref)


_l0_post = pl.pallas_call(
    _l0_body,
    grid=(NB,),
    in_specs=[
        _split_spec(H // 2), _row_spec(1), _row_spec(H),
        _w_spec(H, H), _w_spec(1, H), _w_spec(H, H),
        _w_spec(1, H), _w_spec(1, H), _w_spec(H, OUT),
        _split_spec(H // 2), _row_spec(1), _row_spec(H),
        _w_spec(H, H), _w_spec(1, H), _w_spec(H, H),
        _w_spec(1, H), _w_spec(1, H), _w_spec(H, OUT),
    ],
    out_specs=[
        _row_spec(H), _row_spec(OUT),
        _row_spec(H), _row_spec(OUT),
    ],
    out_shape=[
        jax.ShapeDtypeStruct((N, H), _f32),
        jax.ShapeDtypeStruct((N, OUT), _f32),
        jax.ShapeDtypeStruct((N, H), _f32),
        jax.ShapeDtypeStruct((N, OUT), _f32),
    ],
)


def _l1_half(s2_ref, cnt_ref, h1_ref, Wr_ref, bl_ref, g_ref, b_ref, out_ref):
  inv = 1.0 / jnp.maximum(cnt_ref[...], 1.0)
  agg = jnp.concatenate([s2_ref[0], s2_ref[1]], axis=-1) * inv
  o = agg + bl_ref[...] + _dot(h1_ref[...], Wr_ref[...])
  out_ref[...] = _ln(o, g_ref[...], b_ref[...])


def _l1_body(su2_ref, cntu_ref, h1u_ref, iuWr_ref, iubl_ref, n1ug_ref,
             n1ub_ref,
             si2_ref, cnti_ref, h1i_ref, uiWr_ref, uibl_ref, n1ig_ref,
             n1ib_ref,
             outu_ref, outi_ref):
  _l1_half(su2_ref, cntu_ref, h1u_ref, iuWr_ref, iubl_ref, n1ug_ref,
           n1ub_ref, outu_ref)
  _l1_half(si2_ref, cnti_ref, h1i_ref, uiWr_ref, uibl_ref, n1ig_ref,
           n1ib_ref, outi_ref)


_l1_post = pl.pallas_call(
    _l1_body,
    grid=(NB,),
    in_specs=[
        _split_spec(OUT // 2), _row_spec(1), _row_spec(H),
        _w_spec(H, OUT), _w_spec(1, OUT), _w_spec(1, OUT), _w_spec(1, OUT),
        _split_spec(OUT // 2), _row_spec(1), _row_spec(H),
        _w_spec(H, OUT), _w_spec(1, OUT), _w_spec(1, OUT), _w_spec(1, OUT),
    ],
    out_specs=[_row_spec(OUT), _row_spec(OUT)],
    out_shape=[
        jax.ShapeDtypeStruct((N, OUT), _f32),
        jax.ShapeDtypeStruct((N, OUT), _f32),
    ],
)


@jax.jit
def kernel(x_user, x_item, edge_index_ui, edge_index_iu, proj_user_W,
           proj_user_b, proj_item_W, proj_item_b, l0_ui_Wl, l0_ui_bl,
           l0_ui_Wr, l0_iu_Wl, l0_iu_bl, l0_iu_Wr, ln0_user_g, ln0_user_b,
           ln0_item_g, ln0_item_b, l1_ui_Wl, l1_ui_bl, l1_ui_Wr, l1_iu_Wl,
           l1_iu_bl, l1_iu_Wr, ln1_user_g, ln1_user_b, ln1_item_g,
           ln1_item_b):
  pad = jnp.arange(PAD, dtype=jnp.int32)
  pad_src = pad % W
  pad_dst = N + (pad % 8)

  def _prep(ei):
    src = jnp.concatenate([ei[0], pad_src])
    dst = jnp.concatenate([ei[1], pad_dst]).reshape(NQP, W)
    src2 = jnp.stack([2 * src, 2 * src + 1]).reshape(NC, NQP, W)
    return src2, dst

  src2_ui, dst_ui = _prep(edge_index_ui)
  src2_iu, dst_iu = _prep(edge_index_iu)
  dst2 = jnp.stack([dst_ui, dst_iu])

  zrows_h = jnp.zeros((STR, H // 2), _f32)
  zrows_o = jnp.zeros((STR, OUT // 2), _f32)
  zcnt = jnp.zeros((STR, _CW), _f32)
  ones = jnp.ones((W, _CW), _f32)

  (cnt2,) = _cnt_kernel(dst2, zcnt, ones)
  cnt_i = cnt2[0, :, 0:1]
  cnt_u = cnt2[1, :, 0:1]

  hu, hi = _proj(x_user, x_item, proj_user_W, proj_user_b[None, :],
                 proj_item_W, proj_item_b[None, :])

  (sums_i2,) = _segsum_h(hu.reshape(NC * N, H // 2), src2_ui, dst_ui,
                         zrows_h)
  (sums_u2,) = _segsum_h(hi.reshape(NC * N, H // 2), src2_iu, dst_iu,
                         zrows_h)

  h1u, mu, h1i, mi = _l0_post(
      sums_u2, cnt_u, hu, l0_iu_Wl, l0_iu_bl[None, :], l0_iu_Wr,
      ln0_user_g[None, :], ln0_user_b[None, :], l1_ui_Wl,
      sums_i2, cnt_i, hi, l0_ui_Wl, l0_ui_bl[None, :], l0_ui_Wr,
      ln0_item_g[None, :], ln0_item_b[None, :], l1_iu_Wl)

  (s32_i2,) = _segsum_o(mu.reshape(NC * N, OUT // 2), src2_ui, dst_ui,
                        zrows_o)
  (s32_u2,) = _segsum_o(mi.reshape(NC * N, OUT // 2), src2_iu, dst_iu,
                        zrows_o)

  out_u, out_i = _l1_post(
      s32_u2, cnt_u, h1u, l1_iu_Wr, l1_iu_bl[None, :],
      ln1_user_g[None, :], ln1_user_b[None, :],
      s32_i2, cnt_i, h1i, l1_ui_Wr, l1_ui_bl[None, :],
      ln1_item_g[None, :], ln1_item_b[None, :])
  return out_u, out_i


# speedup vs baseline: 7.2267x; 1.0265x over previous
import functools

import jax
import jax.numpy as jnp
from jax import lax
from jax.experimental import pallas as pl
from jax.experimental.pallas import tpu as pltpu
from jax.experimental.pallas import tpu_sc as plsc

N = 50000
E = 800000
D_IN = 128
H = 64
OUT = 32
EPS = 1e-5

W = 128
NQP = 6252
EP = NQP * W
PAD = EP - E
NS = 16
NC = 2
NA = N + 8
STR = 3128

_f32 = jnp.float32


_SC_PARAMS = pltpu.CompilerParams(use_tc_tiling_on_sc=False)
_MESH = plsc.VectorSubcoreMesh(core_axis_name="c", subcore_axis_name="s")


def _zero_acc(zrows_ref, acc, s):
  @pl.when(s < NS - 1)
  def _():
    pltpu.sync_copy(zrows_ref, acc.at[pl.ds(s * STR, STR)])

  @pl.when(s == NS - 1)
  def _():
    last = NA - (NS - 1) * STR
    pltpu.sync_copy(zrows_ref.at[pl.ds(0, last)],
                    acc.at[pl.ds((NS - 1) * STR, last)])


def _dump_acc(acc, out2d, s):
  @pl.when(s < NS - 1)
  def _():
    pltpu.sync_copy(acc.at[pl.ds(s * STR, STR)], out2d.at[pl.ds(s * STR, STR)])

  @pl.when(s == NS - 1)
  def _():
    last = N - (NS - 1) * STR
    pltpu.sync_copy(acc.at[pl.ds((NS - 1) * STR, last)],
                    out2d.at[pl.ds((NS - 1) * STR, last)])


def _make_segsum(d2, k, w):
  nsup = EP // w // k
  out_type = [jax.ShapeDtypeStruct((NC, N, d2), _f32)]
  scratch = [
      pltpu.VMEM((2, k, w), jnp.int32),
      pltpu.VMEM((2, k, w), jnp.int32),
      pltpu.VMEM((2, k * w, d2), _f32),
      pltpu.VMEM_SHARED((NA, d2), _f32),
      pltpu.SemaphoreType.DMA,
      pltpu.SemaphoreType.DMA,
      pltpu.SemaphoreType.DMA,
  ]

  def body(table_ref, src2_ref, dst_ref, zrows_ref, out_ref,
           srcv, dstv, rows, acc, semi, semg, sems):
    c = lax.axis_index("c")
    s = lax.axis_index("s")
    _zero_acc(zrows_ref, acc, s)
    plsc.subcore_barrier()

    nt = (nsup - s + NS - 1) // NS

    def idx_copies(p, q):
      return [
          pltpu.make_async_copy(src2_ref.at[c].at[pl.ds(q * k, k)],
                                srcv.at[p], semi),
          pltpu.make_async_copy(dst_ref.at[pl.ds(q * k, k)],
                                dstv.at[p], semi),
      ]

    def gather_copies(p):
      return [
          pltpu.make_async_copy(table_ref.at[srcv.at[p].at[j]],
                                rows.at[p].at[pl.ds(j * w, w)], semg)
          for j in range(k)
      ]

    def scatter_copies(p):
      return [
          pltpu.make_async_copy(rows.at[p].at[pl.ds(j * w, w)],
                                acc.at[dstv.at[p].at[j]], sems)
          for j in range(k)
      ]

    @pl.when(nt > 0)
    def _():
      for x in idx_copies(0, s):
        x.start()

    def trip(t, u):
      @pl.when(t < nt)
      def _():
        q = s + t * NS
        for x in idx_copies(u, q):
          x.wait()

        @pl.when(t >= 2)
        def _():
          for x in scatter_copies(u):
            x.wait()

        gs = gather_copies(u)
        for x in gs:
          x.start()

        @pl.when(t + 1 < nt)
        def _():
          for x in idx_copies(1 - u, q + NS):
            x.start()

        sc = scatter_copies(u)
        for g, x in zip(gs, sc):
          g.wait()
          x.start(add=True)

    def pair(r, carry):
      trip(2 * r, 0)
      trip(2 * r + 1, 1)
      return carry

    lax.fori_loop(0, (nt + 1) // 2, pair, 0)

    @pl.when(nt >= 2)
    def _():
      for x in scatter_copies(0) + scatter_copies(1):
        x.wait()

    @pl.when(nt == 1)
    def _():
      for x in scatter_copies(0):
        x.wait()

    plsc.subcore_barrier()
    _dump_acc(acc, out_ref.at[c], s)

  return pl.kernel(
      body, out_type=out_type, mesh=_MESH, scratch_types=scratch,
      compiler_params=_SC_PARAMS)


_WH = W
_WO = W
_segsum_h = _make_segsum(H // 2, 3, _WH)
_segsum_o = _make_segsum(OUT // 2, 12, _WO)

_KC = 12
_CW = 16


def _cnt_body(dst2_ref, zcnt_ref, ones_ref, out_ref, dstv, onesv, cntacc,
              semc):
  c = lax.axis_index("c")
  s = lax.axis_index("s")

  @pl.when(s < NS - 1)
  def _():
    pltpu.sync_copy(zcnt_ref, cntacc.at[pl.ds(s * STR, STR)])

  @pl.when(s == NS - 1)
  def _():
    last = NA - (NS - 1) * STR
    pltpu.sync_copy(zcnt_ref.at[pl.ds(0, last)],
                    cntacc.at[pl.ds((NS - 1) * STR, last)])

  pltpu.sync_copy(ones_ref, onesv)
  plsc.subcore_barrier()

  nsup = NQP // _KC

  nt = (nsup - s + NS - 1) // NS

  def step(t, carry):
    q = s + t * NS
    pltpu.sync_copy(dst2_ref.at[c].at[pl.ds(q * _KC, _KC)], dstv)
    cs = [
        pltpu.make_async_copy(onesv, cntacc.at[dstv.at[j]], semc)
        for j in range(_KC)
    ]
    for x in cs:
      x.start(add=True)
    for x in cs:
      x.wait()
    return carry

  lax.fori_loop(0, nt, step, 0)
  plsc.subcore_barrier()
  _dump_acc(cntacc, out_ref.at[c], s)


_cnt_kernel = pl.kernel(
    _cnt_body,
    out_type=[jax.ShapeDtypeStruct((NC, N, _CW), _f32)],
    mesh=_MESH,
    scratch_types=[
        pltpu.VMEM((_KC, W), jnp.int32),
        pltpu.VMEM((W, _CW), _f32),
        pltpu.VMEM_SHARED((NA, _CW), _f32),
        pltpu.SemaphoreType.DMA,
    ],
    compiler_params=_SC_PARAMS)


R = 1000
NB = N // R

_PREC = lax.Precision.HIGHEST


def _dot(a, b):
  return jnp.dot(a, b, precision=_PREC, preferred_element_type=_f32)


def _ln(h, g, b):
  m = jnp.mean(h, axis=-1, keepdims=True)
  v = jnp.mean((h - m) * (h - m), axis=-1, keepdims=True)
  return (h - m) * lax.rsqrt(v + EPS) * g + b


def _row_spec(d):
  return pl.BlockSpec((R, d), lambda i: (i, 0))


def _split_spec(d):
  return pl.BlockSpec((2, R, d), lambda i: (0, i, 0))


def _w_spec(a, b):
  return pl.BlockSpec((a, b), lambda i: (0, 0))


def _proj_body(xu_ref, xi_ref, puW_ref, pub_ref, piW_ref, pib_ref,
               hu_ref, hi_ref):
  hu_ref[...] = jnp.maximum(_dot(xu_ref[...], puW_ref[...]) + pub_ref[...],
                            0.0)
  hi_ref[...] = jnp.maximum(_dot(xi_ref[...], piW_ref[...]) + pib_ref[...],
                            0.0)


_proj = pl.pallas_call(
    _proj_body,
    grid=(NB,),
    in_specs=[
        _row_spec(D_IN), _row_spec(D_IN),
        _w_spec(D_IN, H), _w_spec(1, H),
        _w_spec(D_IN, H), _w_spec(1, H),
    ],
    out_specs=[_row_spec(H), _row_spec(H)],
    out_shape=[
        jax.ShapeDtypeStruct((N, H), _f32),
        jax.ShapeDtypeStruct((N, H), _f32),
    ],
)


def _l0_half(s2_ref, cnt_ref, h_ref, Wl_ref, bl_ref, Wr_ref, g_ref, b_ref,
             Wnext_ref, h1_ref, m_ref):
  inv = 1.0 / jnp.maximum(cnt_ref[...], 1.0)
  agg = jnp.concatenate([s2_ref[0], s2_ref[1]], axis=-1) * inv
  o = _dot(agg, Wl_ref[...]) + bl_ref[...] + _dot(h_ref[...], Wr_ref[...])
  h1 = jnp.maximum(_ln(o, g_ref[...], b_ref[...]), 0.0)
  h1_ref[...] = h1
  m_ref[...] = _dot(h1, Wnext_ref[...])


def _l0_body(su2_ref, cntu_ref, hu_ref, iuWl_ref, iubl_ref, iuWr_ref,
             n0ug_ref, n0ub_ref, uiWl1_ref,
             si2_ref, cnti_ref, hi_ref, uiWl_ref, uibl_ref, uiWr_ref,
             n0ig_ref, n0ib_ref, iuWl1_ref,
             h1u_ref, mu_ref, h1i_ref, mi_ref):
  _l0_half(su2_ref, cntu_ref, hu_ref, iuWl_ref, iubl_ref, iuWr_ref,
           n0ug_ref, n0ub_ref, uiWl1_ref, h1u_ref, mu_ref)
  _l0_half(si2_ref, cnti_ref, hi_ref, uiWl_ref, uibl_ref, uiWr_ref,
           n0ig_ref, n0ib_ref, iuWl1_ref, h1i_ref, mi_ref)


_l0_post = pl.pallas_call(
    _l0_body,
    grid=(NB,),
    in_specs=[
        _split_spec(H // 2), _row_spec(1), _row_spec(H),
        _w_spec(H, H), _w_spec(1, H), _w_spec(H, H),
        _w_spec(1, H), _w_spec(1, H), _w_spec(H, OUT),
        _split_spec(H // 2), _row_spec(1), _row_spec(H),
        _w_spec(H, H), _w_spec(1, H), _w_spec(H, H),
        _w_spec(1, H), _w_spec(1, H), _w_spec(H, OUT),
    ],
    out_specs=[
        _row_spec(H), _row_spec(OUT),
        _row_spec(H), _row_spec(OUT),
    ],
    out_shape=[
        jax.ShapeDtypeStruct((N, H), _f32),
        jax.ShapeDtypeStruct((N, OUT), _f32),
        jax.ShapeDtypeStruct((N, H), _f32),
        jax.ShapeDtypeStruct((N, OUT), _f32),
    ],
)


def _l1_half(s2_ref, cnt_ref, h1_ref, Wr_ref, bl_ref, g_ref, b_ref, out_ref):
  inv = 1.0 / jnp.maximum(cnt_ref[...], 1.0)
  agg = jnp.concatenate([s2_ref[0], s2_ref[1]], axis=-1) * inv
  o = agg + bl_ref[...] + _dot(h1_ref[...], Wr_ref[...])
  out_ref[...] = _ln(o, g_ref[...], b_ref[...])


def _l1_body(su2_ref, cntu_ref, h1u_ref, iuWr_ref, iubl_ref, n1ug_ref,
             n1ub_ref,
             si2_ref, cnti_ref, h1i_ref, uiWr_ref, uibl_ref, n1ig_ref,
             n1ib_ref,
             outu_ref, outi_ref):
  _l1_half(su2_ref, cntu_ref, h1u_ref, iuWr_ref, iubl_ref, n1ug_ref,
           n1ub_ref, outu_ref)
  _l1_half(si2_ref, cnti_ref, h1i_ref, uiWr_ref, uibl_ref, n1ig_ref,
           n1ib_ref, outi_ref)


_l1_post = pl.pallas_call(
    _l1_body,
    grid=(NB,),
    in_specs=[
        _split_spec(OUT // 2), _row_spec(1), _row_spec(H),
        _w_spec(H, OUT), _w_spec(1, OUT), _w_spec(1, OUT), _w_spec(1, OUT),
        _split_spec(OUT // 2), _row_spec(1), _row_spec(H),
        _w_spec(H, OUT), _w_spec(1, OUT), _w_spec(1, OUT), _w_spec(1, OUT),
    ],
    out_specs=[_row_spec(OUT), _row_spec(OUT)],
    out_shape=[
        jax.ShapeDtypeStruct((N, OUT), _f32),
        jax.ShapeDtypeStruct((N, OUT), _f32),
    ],
)


@jax.jit
def kernel(x_user, x_item, edge_index_ui, edge_index_iu, proj_user_W,
           proj_user_b, proj_item_W, proj_item_b, l0_ui_Wl, l0_ui_bl,
           l0_ui_Wr, l0_iu_Wl, l0_iu_bl, l0_iu_Wr, ln0_user_g, ln0_user_b,
           ln0_item_g, ln0_item_b, l1_ui_Wl, l1_ui_bl, l1_ui_Wr, l1_iu_Wl,
           l1_iu_bl, l1_iu_Wr, ln1_user_g, ln1_user_b, ln1_item_g,
           ln1_item_b):
  pad = jnp.arange(PAD, dtype=jnp.int32)
  pad_src = pad % W
  pad_dst = N + (pad % 8)

  def _prep(ei):
    src = jnp.concatenate([ei[0], pad_src])
    dst = jnp.concatenate([ei[1], pad_dst])
    src2 = jnp.stack([2 * src, 2 * src + 1])
    return src2, dst

  src2_ui, dst_ui = _prep(edge_index_ui)
  src2_iu, dst_iu = _prep(edge_index_iu)
  dst2 = jnp.stack([dst_ui.reshape(NQP, W), dst_iu.reshape(NQP, W)])

  def _rs(src2, dst, w):
    return src2.reshape(NC, EP // w, w), dst.reshape(EP // w, w)

  src2_ui_h, dst_ui_h = _rs(src2_ui, dst_ui, _WH)
  src2_iu_h, dst_iu_h = _rs(src2_iu, dst_iu, _WH)
  src2_ui_o, dst_ui_o = _rs(src2_ui, dst_ui, _WO)
  src2_iu_o, dst_iu_o = _rs(src2_iu, dst_iu, _WO)

  zrows_h = jnp.zeros((STR, H // 2), _f32)
  zrows_o = jnp.zeros((STR, OUT // 2), _f32)
  zcnt = jnp.zeros((STR, _CW), _f32)
  ones = jnp.ones((W, _CW), _f32)

  (cnt2,) = _cnt_kernel(dst2, zcnt, ones)
  cnt_i = cnt2[0, :, 0:1]
  cnt_u = cnt2[1, :, 0:1]

  hu, hi = _proj(x_user, x_item, proj_user_W, proj_user_b[None, :],
                 proj_item_W, proj_item_b[None, :])

  (sums_i2,) = _segsum_h(hu.reshape(NC * N, H // 2), src2_ui_h, dst_ui_h,
                         zrows_h)
  (sums_u2,) = _segsum_h(hi.reshape(NC * N, H // 2), src2_iu_h, dst_iu_h,
                         zrows_h)

  h1u, mu, h1i, mi = _l0_post(
      sums_u2, cnt_u, hu, l0_iu_Wl, l0_iu_bl[None, :], l0_iu_Wr,
      ln0_user_g[None, :], ln0_user_b[None, :], l1_ui_Wl,
      sums_i2, cnt_i, hi, l0_ui_Wl, l0_ui_bl[None, :], l0_ui_Wr,
      ln0_item_g[None, :], ln0_item_b[None, :], l1_iu_Wl)

  (s32_i2,) = _segsum_o(mu.reshape(NC * N, OUT // 2), src2_ui_o, dst_ui_o,
                        zrows_o)
  (s32_u2,) = _segsum_o(mi.reshape(NC * N, OUT // 2), src2_iu_o, dst_iu_o,
                        zrows_o)

  out_u, out_i = _l1_post(
      s32_u2, cnt_u, h1u, l1_iu_Wr, l1_iu_bl[None, :],
      ln1_user_g[None, :], ln1_user_b[None, :],
      s32_i2, cnt_i, h1i, l1_ui_Wr, l1_ui_bl[None, :],
      ln1_item_g[None, :], ln1_item_b[None, :])
  return out_u, out_i
